# baseline (speedup 1.0000x reference)
import jax
import jax.numpy as jnp
from jax.experimental import pallas as pl
from jax.experimental.pallas import tpu as pltpu

_S = 4
_H = 64
_LANE = 128
_TILE = 262144
_SUB = 8192


def _mlp_kernel(xv_ref, w1_ref, w2_ref, w3_ref, o_ref):
    rows = xv_ref.shape[0]
    tb = rows * (_LANE // _S)
    srows = _SUB // (_LANE // _S)
    for s in range(tb // _SUB):
        xb = xv_ref[s * srows:(s + 1) * srows, :]
        xt = (
            xb.reshape(srows // 8, 2, _S, _LANE)
            .transpose(2, 0, 1, 3)
            .reshape(_S, _SUB)
        )
        ones = jnp.ones((1, _SUB), jnp.float32)
        h1 = jnp.maximum(
            jnp.dot(
                w1_ref[...],
                jnp.concatenate([xt, ones], axis=0),
                preferred_element_type=jnp.float32,
            ),
            0.0,
        )
        h2 = jnp.maximum(
            jnp.dot(
                w2_ref[...],
                jnp.concatenate([h1, ones], axis=0),
                preferred_element_type=jnp.float32,
            ),
            0.0,
        )
        h3 = jnp.dot(
            w3_ref[...],
            jnp.concatenate([h2, ones], axis=0),
            preferred_element_type=jnp.float32,
        )
        o_ref[s * (_SUB // 64):(s + 1) * (_SUB // 64), :] = (
            h3[:2]
            .reshape(2, _SUB // _LANE, _LANE)
            .transpose(1, 0, 2)
            .reshape(_SUB // 64, _LANE)
        )


def kernel(x, w1, b1, w2, b2, w3p, b3p):
    batch = x.shape[0]
    tile = _TILE
    padded = ((batch + tile - 1) // tile) * tile
    if padded != batch:
        x = jnp.pad(x, ((0, padded - batch), (0, 0)))
    xv = (
        x.reshape(padded // _LANE, _LANE, _S)
        .transpose(0, 2, 1)
        .reshape(padded * _S // _LANE, _LANE)
    )
    w1a = jnp.concatenate([w1, b1], axis=1)
    w2a = jnp.concatenate([w2, b2], axis=1)
    w3a = jnp.concatenate([w3p, b3p], axis=1)
    grid = (padded // tile,)
    flops = 2 * padded * (_S * _H + _H * _H + _H * 8)
    bytes_accessed = 4 * (padded * _S + padded * 2 + _H * _S + _H * _H + 8 * _H)
    out = pl.pallas_call(
        _mlp_kernel,
        out_shape=jax.ShapeDtypeStruct((padded * 2 // _LANE, _LANE), jnp.float32),
        grid_spec=pltpu.PrefetchScalarGridSpec(
            num_scalar_prefetch=0,
            grid=grid,
            in_specs=[
                pl.BlockSpec((tile // (_LANE // _S), _LANE), lambda i: (i, 0)),
                pl.BlockSpec((_H, _S + 1), lambda i: (0, 0)),
                pl.BlockSpec((_H, _H + 1), lambda i: (0, 0)),
                pl.BlockSpec((8, _H + 1), lambda i: (0, 0)),
            ],
            out_specs=pl.BlockSpec((tile // 64, _LANE), lambda i: (i, 0)),
        ),
        compiler_params=pltpu.CompilerParams(
            dimension_semantics=("parallel",),
        ),
        cost_estimate=pl.CostEstimate(
            flops=flops, transcendentals=0, bytes_accessed=bytes_accessed
        ),
    )(xv, w1a, w2a, w3a)
    res = (
        out.reshape(padded // _LANE, 2, _LANE)
        .transpose(0, 2, 1)
        .reshape(padded, 2)
    )
    return res[:batch] if padded != batch else res


# speedup vs baseline: 1.0162x; 1.0162x over previous
import jax
import jax.numpy as jnp
from jax.experimental import pallas as pl
from jax.experimental.pallas import tpu as pltpu

_S = 4
_H = 64
_LANE = 128
_TILE = 262144
_SUB = 16384


def _mlp_kernel(xv_ref, w1_ref, w2_ref, w3_ref, o_ref):
    rows = xv_ref.shape[0]
    tb = rows * (_LANE // _S)
    srows = _SUB // (_LANE // _S)
    for s in range(tb // _SUB):
        xb = xv_ref[s * srows:(s + 1) * srows, :]
        xt = (
            xb.reshape(srows // 8, 2, _S, _LANE)
            .transpose(2, 0, 1, 3)
            .reshape(_S, _SUB)
        )
        ones = jnp.ones((1, _SUB), jnp.float32)
        h1 = jnp.maximum(
            jnp.dot(
                w1_ref[...],
                jnp.concatenate([xt, ones], axis=0),
                preferred_element_type=jnp.float32,
            ),
            0.0,
        )
        h2 = jnp.maximum(
            jnp.dot(
                w2_ref[...],
                jnp.concatenate([h1, ones], axis=0),
                preferred_element_type=jnp.float32,
            ),
            0.0,
        )
        h3 = jnp.dot(
            w3_ref[...],
            jnp.concatenate([h2, ones], axis=0),
            preferred_element_type=jnp.float32,
        )
        o_ref[s * (_SUB // 64):(s + 1) * (_SUB // 64), :] = (
            h3[:2]
            .reshape(2, _SUB // _LANE, _LANE)
            .transpose(1, 0, 2)
            .reshape(_SUB // 64, _LANE)
        )


def kernel(x, w1, b1, w2, b2, w3p, b3p):
    batch = x.shape[0]
    tile = _TILE
    padded = ((batch + tile - 1) // tile) * tile
    if padded != batch:
        x = jnp.pad(x, ((0, padded - batch), (0, 0)))
    xv = (
        x.reshape(padded // _LANE, _LANE, _S)
        .transpose(0, 2, 1)
        .reshape(padded * _S // _LANE, _LANE)
    )
    w1a = jnp.concatenate([w1, b1], axis=1)
    w2a = jnp.concatenate([w2, b2], axis=1)
    w3a = jnp.concatenate([w3p, b3p], axis=1)
    grid = (padded // tile,)
    flops = 2 * padded * (_S * _H + _H * _H + _H * 8)
    bytes_accessed = 4 * (padded * _S + padded * 2 + _H * _S + _H * _H + 8 * _H)
    out = pl.pallas_call(
        _mlp_kernel,
        out_shape=jax.ShapeDtypeStruct((padded * 2 // _LANE, _LANE), jnp.float32),
        grid_spec=pltpu.PrefetchScalarGridSpec(
            num_scalar_prefetch=0,
            grid=grid,
            in_specs=[
                pl.BlockSpec((tile // (_LANE // _S), _LANE), lambda i: (i, 0)),
                pl.BlockSpec((_H, _S + 1), lambda i: (0, 0)),
                pl.BlockSpec((_H, _H + 1), lambda i: (0, 0)),
                pl.BlockSpec((8, _H + 1), lambda i: (0, 0)),
            ],
            out_specs=pl.BlockSpec((tile // 64, _LANE), lambda i: (i, 0)),
        ),
        compiler_params=pltpu.CompilerParams(
            dimension_semantics=("parallel",),
        ),
        cost_estimate=pl.CostEstimate(
            flops=flops, transcendentals=0, bytes_accessed=bytes_accessed
        ),
    )(xv, w1a, w2a, w3a)
    res = (
        out.reshape(padded // _LANE, 2, _LANE)
        .transpose(0, 2, 1)
        .reshape(padded, 2)
    )
    return res[:batch] if padded != batch else res
